# baseline (speedup 1.0000x reference)
import functools

import jax
import jax.numpy as jnp
from jax import lax
from jax.experimental import pallas as pl
from jax.experimental.pallas import tpu as pltpu
from jax.experimental.pallas import tpu_sc as plsc

_LQ = 4096
_LK = 4096
_WIN = 5120


def _body(table_hbm, delta_hbm, out_hbm, table_v, delta_v, win_v, sem):
    cid = lax.axis_index("c")
    sid = lax.axis_index("s")
    wid = sid * 2 + cid
    residue = wid % 8
    m0 = (wid // 8) * 128

    pltpu.sync_copy(table_hbm, table_v)
    pltpu.sync_copy(delta_hbm, delta_v)

    iot = lax.broadcasted_iota(jnp.int32, (16,), 0)
    c0 = iot + (-888 - residue - 8 * m0) + delta_v[...]

    def build(tb, carry):
        idx = jnp.clip(c0 + tb * 16, 0, 256)
        win_v[pl.ds(pl.multiple_of(tb * 16, 16), 16)] = \
            plsc.load_gather(table_v, [idx])
        return carry

    lax.fori_loop(0, _WIN // 16, build, 0)

    i_base = residue + 8 * m0

    def rows(blk, carry):
        for j in range(8):
            src_off = pl.multiple_of(1016 - 64 * blk - 8 * j, 8)
            dst_off = pl.multiple_of(
                (i_base + 64 * blk + 8 * j) * _LK, 8)
            pltpu.async_copy(
                win_v.at[pl.ds(src_off, _LK)],
                out_hbm.at[pl.ds(dst_off, _LK)], sem)
        return carry

    lax.fori_loop(0, 16, rows, 0)

    def drain(blk, carry):
        for _ in range(8):
            pltpu.make_async_copy(
                win_v.at[pl.ds(0, _LK)], out_hbm.at[pl.ds(0, _LK)], sem
            ).wait()
        return carry

    lax.fori_loop(0, 16, drain, 0)


@functools.partial(jax.jit)
def _rel_pos_sc(table_p, delta_arr):
    mesh = plsc.VectorSubcoreMesh(core_axis_name="c", subcore_axis_name="s")
    return pl.kernel(
        _body,
        out_type=jax.ShapeDtypeStruct((_LQ * _LK,), jnp.float32),
        mesh=mesh,
        compiler_params=pltpu.CompilerParams(needs_layout_passes=False),
        scratch_types=[
            pltpu.VMEM((272,), jnp.float32),
            pltpu.VMEM((16,), jnp.int32),
            pltpu.VMEM((_WIN,), jnp.float32),
            pltpu.SemaphoreType.DMA,
        ],
    )(table_p, delta_arr)


def kernel(embeddings_table, length_q, length_k):
    delta = (length_k - _LK) - (length_q - _LQ)
    table_p = jnp.pad(embeddings_table.astype(jnp.float32), (0, 15))
    delta_arr = jnp.full((16,), delta, dtype=jnp.int32)
    return _rel_pos_sc(table_p, delta_arr).reshape(_LQ, _LK)


# speedup vs baseline: 1.1939x; 1.1939x over previous
import functools

import jax
import jax.numpy as jnp
from jax import lax
from jax.experimental import pallas as pl
from jax.experimental.pallas import tpu as pltpu
from jax.experimental.pallas import tpu_sc as plsc

_LQ = 4096
_LK = 4096
_WIN = 5120


def _body(table_hbm, delta_hbm, out_hbm, table_v, delta_v, win_v, sem):
    cid = lax.axis_index("c")
    sid = lax.axis_index("s")
    wid = sid * 2 + cid
    residue = wid % 8
    m0 = (wid // 8) * 128

    pltpu.sync_copy(table_hbm, table_v)
    pltpu.sync_copy(delta_hbm, delta_v)

    iot = lax.broadcasted_iota(jnp.int32, (16,), 0)
    c0 = iot + (-888 - residue - 8 * m0) + delta_v[...]

    def build(tb, carry):
        idx = jnp.clip(c0 + tb * 16, 0, 256)
        win_v[pl.ds(pl.multiple_of(tb * 16, 16), 16)] = \
            plsc.load_gather(table_v, [idx])
        return carry

    lax.fori_loop(0, _WIN // 16, build, 0)

    i_base = residue + 8 * m0

    def rows(blk, carry):
        for j in range(8):
            src_off = pl.multiple_of(1016 - 64 * blk - 8 * j, 8)
            dst_off = pl.multiple_of(
                (i_base + 64 * blk + 8 * j) * _LK, 8)
            pltpu.async_copy(
                win_v.at[pl.ds(src_off, _LK)],
                out_hbm.at[pl.ds(dst_off, _LK)], sem)
        return carry

    lax.fori_loop(0, 16, rows, 0)

    def drain(blk, carry):
        for _ in range(8):
            pltpu.make_async_copy(
                win_v.at[pl.ds(0, _LK)], out_hbm.at[pl.ds(0, _LK)], sem
            ).wait()
        return carry

    lax.fori_loop(0, 16, drain, 0)


@functools.partial(jax.jit)
def _rel_pos_sc(table_p, delta_arr):
    mesh = plsc.VectorSubcoreMesh(core_axis_name="c", subcore_axis_name="s")
    return pl.kernel(
        _body,
        out_type=jax.ShapeDtypeStruct((_LQ * _LK,), jnp.float32),
        mesh=mesh,
        compiler_params=pltpu.CompilerParams(needs_layout_passes=False),
        scratch_types=[
            pltpu.VMEM((272,), jnp.float32),
            pltpu.VMEM((16,), jnp.int32),
            pltpu.VMEM((_WIN,), jnp.float32),
            pltpu.SemaphoreType.DMA,
        ],
    )(table_p, delta_arr)


_BR = 128


def _retile_body(in_ref, out_ref):
    out_ref[...] = in_ref[...].reshape(_BR, _LK)


@jax.jit
def _retile(x3):
    return pl.pallas_call(
        _retile_body,
        grid=(_LQ // _BR,),
        in_specs=[pl.BlockSpec((_BR, _LK // 128, 128), lambda i: (i, 0, 0))],
        out_specs=pl.BlockSpec((_BR, _LK), lambda i: (i, 0)),
        out_shape=jax.ShapeDtypeStruct((_LQ, _LK), jnp.float32),
    )(x3)


def kernel(embeddings_table, length_q, length_k):
    delta = (length_k - _LK) - (length_q - _LQ)
    table_p = jnp.pad(embeddings_table.astype(jnp.float32), (0, 15))
    delta_arr = jnp.full((16,), delta, dtype=jnp.int32)
    flat = _rel_pos_sc(table_p, delta_arr)
    return _retile(flat.reshape(_LQ, _LK // 128, 128))


# speedup vs baseline: 1.2720x; 1.0654x over previous
import functools

import jax
import jax.numpy as jnp
from jax import lax
from jax.experimental import pallas as pl
from jax.experimental.pallas import tpu as pltpu
from jax.experimental.pallas import tpu_sc as plsc

_LQ = 4096
_LK = 4096
_WIN = 5120


def _body(table_hbm, delta_hbm, out_hbm, table_v, delta_v, win_v, sem):
    cid = lax.axis_index("c")
    sid = lax.axis_index("s")
    wid = sid * 2 + cid
    residue = wid % 8
    m0 = (wid // 8) * 128

    pltpu.sync_copy(table_hbm, table_v)
    pltpu.sync_copy(delta_hbm, delta_v)

    iot = lax.broadcasted_iota(jnp.int32, (16,), 0)
    c0 = iot + (-888 - residue - 8 * m0) + delta_v[...]

    def build(tb, carry):
        idx = jnp.clip(c0 + tb * 16, 0, 256)
        win_v[pl.ds(pl.multiple_of(tb * 16, 16), 16)] = \
            plsc.load_gather(table_v, [idx])
        return carry

    lax.fori_loop(0, _WIN // 16, build, 0)

    i_base = residue + 8 * m0

    def rows(blk, carry):
        for j in range(8):
            src_off = pl.multiple_of(1016 - 64 * blk - 8 * j, 8)
            dst_off = pl.multiple_of(
                (i_base + 64 * blk + 8 * j) * _LK, 8)
            pltpu.async_copy(
                win_v.at[pl.ds(src_off, _LK)],
                out_hbm.at[pl.ds(dst_off, _LK)], sem)
        return carry

    lax.fori_loop(0, 16, rows, 0)

    def drain(blk, carry):
        for _ in range(8):
            pltpu.make_async_copy(
                win_v.at[pl.ds(0, _LK)], out_hbm.at[pl.ds(0, _LK)], sem
            ).wait()
        return carry

    lax.fori_loop(0, 16, drain, 0)


@functools.partial(jax.jit)
def _rel_pos_sc(table_p, delta_arr):
    mesh = plsc.VectorSubcoreMesh(core_axis_name="c", subcore_axis_name="s")
    return pl.kernel(
        _body,
        out_type=jax.ShapeDtypeStruct((_LQ * _LK,), jnp.float32),
        mesh=mesh,
        compiler_params=pltpu.CompilerParams(needs_layout_passes=False),
        scratch_types=[
            pltpu.VMEM((272,), jnp.float32),
            pltpu.VMEM((16,), jnp.int32),
            pltpu.VMEM((_WIN,), jnp.float32),
            pltpu.SemaphoreType.DMA,
        ],
    )(table_p, delta_arr)


_BR = 256


def _retile_body(in_ref, out_ref):
    out_ref[...] = in_ref[...].reshape(_BR, _LK)


@jax.jit
def _retile(x3):
    return pl.pallas_call(
        _retile_body,
        grid=(_LQ // _BR,),
        in_specs=[pl.BlockSpec((_BR, _LK // 128, 128), lambda i: (i, 0, 0))],
        out_specs=pl.BlockSpec((_BR, _LK), lambda i: (i, 0)),
        out_shape=jax.ShapeDtypeStruct((_LQ, _LK), jnp.float32),
    )(x3)


def kernel(embeddings_table, length_q, length_k):
    delta = (length_k - _LK) - (length_q - _LQ)
    table_p = jnp.pad(embeddings_table.astype(jnp.float32), (0, 15))
    delta_arr = jnp.full((16,), delta, dtype=jnp.int32)
    flat = _rel_pos_sc(table_p, delta_arr)
    return _retile(flat.reshape(_LQ, _LK // 128, 128))


# speedup vs baseline: 1.2922x; 1.0159x over previous
import functools

import jax
import jax.numpy as jnp
from jax import lax
from jax.experimental import pallas as pl
from jax.experimental.pallas import tpu as pltpu
from jax.experimental.pallas import tpu_sc as plsc

_LQ = 4096
_LK = 4096
_WIN = 5120


def _body(table_hbm, delta_hbm, out_hbm, table_v, delta_v, win_v, sem):
    cid = lax.axis_index("c")
    sid = lax.axis_index("s")
    wid = sid * 2 + cid
    residue = wid % 8
    m0 = (wid // 8) * 128

    pltpu.sync_copy(table_hbm, table_v)
    pltpu.sync_copy(delta_hbm, delta_v)

    iot = lax.broadcasted_iota(jnp.int32, (16,), 0)
    c0 = iot + (-888 - residue - 8 * m0) + delta_v[...]

    def build(tb, carry):
        idx = jnp.clip(c0 + tb * 16, 0, 256)
        win_v[pl.ds(pl.multiple_of(tb * 16, 16), 16)] = \
            plsc.load_gather(table_v, [idx])
        return carry

    lax.fori_loop(0, _WIN // 16, build, 0)

    i_base = residue + 8 * m0

    def rows(blk, carry):
        for j in range(8):
            src_off = pl.multiple_of(1016 - 64 * blk - 8 * j, 8)
            dst_off = pl.multiple_of(
                (i_base + 64 * blk + 8 * j) * _LK, 8)
            pltpu.async_copy(
                win_v.at[pl.ds(src_off, _LK)],
                out_hbm.at[pl.ds(dst_off, _LK)], sem)
        return carry

    lax.fori_loop(0, 16, rows, 0)

    def drain(blk, carry):
        for _ in range(8):
            pltpu.make_async_copy(
                win_v.at[pl.ds(0, _LK)], out_hbm.at[pl.ds(0, _LK)], sem
            ).wait()
        return carry

    lax.fori_loop(0, 16, drain, 0)


@functools.partial(jax.jit)
def _rel_pos_sc(table_p, delta_arr):
    mesh = plsc.VectorSubcoreMesh(core_axis_name="c", subcore_axis_name="s")
    return pl.kernel(
        _body,
        out_type=jax.ShapeDtypeStruct((_LQ * _LK,), jnp.float32),
        mesh=mesh,
        compiler_params=pltpu.CompilerParams(needs_layout_passes=False),
        scratch_types=[
            pltpu.VMEM((272,), jnp.float32),
            pltpu.VMEM((16,), jnp.int32),
            pltpu.VMEM((_WIN,), jnp.float32),
            pltpu.SemaphoreType.DMA,
        ],
    )(table_p, delta_arr)


_BR = 512


def _retile_body(in_ref, out_ref):
    out_ref[...] = in_ref[...].reshape(_BR, _LK)


@jax.jit
def _retile(x3):
    return pl.pallas_call(
        _retile_body,
        grid=(_LQ // _BR,),
        in_specs=[pl.BlockSpec((_BR, _LK // 128, 128), lambda i: (i, 0, 0))],
        out_specs=pl.BlockSpec((_BR, _LK), lambda i: (i, 0)),
        out_shape=jax.ShapeDtypeStruct((_LQ, _LK), jnp.float32),
    )(x3)


def kernel(embeddings_table, length_q, length_k):
    delta = (length_k - _LK) - (length_q - _LQ)
    table_p = jnp.pad(embeddings_table.astype(jnp.float32), (0, 15))
    delta_arr = jnp.full((16,), delta, dtype=jnp.int32)
    flat = _rel_pos_sc(table_p, delta_arr)
    return _retile(flat.reshape(_LQ, _LK // 128, 128))
